# Initial kernel scaffold; baseline (speedup 1.0000x reference)
import functools

import jax
import jax.numpy as jnp
from jax import lax
from jax.experimental import pallas as pl
from jax.experimental.pallas import tpu as pltpu
from jax.experimental.pallas import tpu_sc as plsc

NC = 2
NS = 16
NW = NC * NS
CHUNK = 32
NBUF = 2


@functools.partial(jax.jit, static_argnames=("n_rows", "d"))
def _sc_embedding_gather(x_flat, table, n_rows, d):
    n_per_w = n_rows // NW
    nchunk = n_per_w // CHUNK
    mesh = plsc.VectorSubcoreMesh(core_axis_name="c", subcore_axis_name="s")

    @functools.partial(
        pl.kernel,
        mesh=mesh,
        out_type=jax.ShapeDtypeStruct((n_rows, d), jnp.float32),
        scratch_types=[
            pltpu.VMEM((nchunk, CHUNK), jnp.int32),
            pltpu.VMEM((CHUNK, d), jnp.float32),
            pltpu.VMEM((CHUNK, d), jnp.float32),
            pltpu.SemaphoreType.DMA,
            pltpu.SemaphoreType.DMA,
        ],
    )
    def run(x_hbm, table_hbm, out_hbm, idx_v, buf0, buf1, sem0, sem1):
        wid = lax.axis_index("s") * NC + lax.axis_index("c")
        base = wid * n_per_w
        pltpu.sync_copy(x_hbm.at[wid], idx_v)
        bufs = (buf0, buf1)
        sems = (sem0, sem1)
        for b in range(NBUF):
            pltpu.async_copy(table_hbm.at[idx_v.at[b]], bufs[b], sems[b])

        def outer(g2, carry):
            for b in range(NBUF):
                g = g2 * NBUF + b
                pltpu.make_async_copy(
                    table_hbm.at[idx_v.at[g]], bufs[b], sems[b]
                ).wait()
                pltpu.sync_copy(bufs[b], out_hbm.at[pl.ds(base + g * CHUNK, CHUNK)])

                @pl.when(g + NBUF < nchunk)
                def _():
                    pltpu.async_copy(
                        table_hbm.at[idx_v.at[g + NBUF]], bufs[b], sems[b]
                    )

            return carry

        lax.fori_loop(0, nchunk // NBUF, outer, 0)

    return run(x_flat, table)


def kernel(x, token_table):
    batch, seq = x.shape
    vocab, d = token_table.shape
    n_rows = batch * seq
    x_flat = x.reshape(NW, n_rows // NW // CHUNK, CHUNK).astype(jnp.int32)
    out = _sc_embedding_gather(x_flat, token_table, n_rows, d)
    return out.reshape(batch, seq, d)


# speedup vs baseline: 1.0352x; 1.0352x over previous
import functools

import jax
import jax.numpy as jnp
from jax import lax
from jax.experimental import pallas as pl
from jax.experimental.pallas import tpu as pltpu
from jax.experimental.pallas import tpu_sc as plsc

NC = 2
NS = 16
NW = NC * NS
CHUNK = 32
NBUF = 2


@functools.partial(jax.jit, static_argnames=("n_rows", "d"))
def _sc_embedding_gather(x_flat, table, n_rows, d):
    n_per_w = n_rows // NW
    nchunk = n_per_w // CHUNK
    mesh = plsc.VectorSubcoreMesh(core_axis_name="c", subcore_axis_name="s")

    @functools.partial(
        pl.kernel,
        mesh=mesh,
        compiler_params=pltpu.CompilerParams(use_tc_tiling_on_sc=False),
        out_type=jax.ShapeDtypeStruct((n_rows, d), jnp.float32),
        scratch_types=[
            pltpu.VMEM((nchunk, CHUNK), jnp.int32),
            pltpu.VMEM((CHUNK, d), jnp.float32),
            pltpu.VMEM((CHUNK, d), jnp.float32),
            pltpu.SemaphoreType.DMA,
            pltpu.SemaphoreType.DMA,
        ],
    )
    def run(x_hbm, table_hbm, out_hbm, idx_v, buf0, buf1, sem0, sem1):
        wid = lax.axis_index("s") * NC + lax.axis_index("c")
        base = wid * n_per_w
        pltpu.sync_copy(x_hbm.at[wid], idx_v)
        bufs = (buf0, buf1)
        sems = (sem0, sem1)
        for b in range(NBUF):
            pltpu.async_copy(table_hbm.at[idx_v.at[b]], bufs[b], sems[b])

        def outer(g2, carry):
            for b in range(NBUF):
                g = g2 * NBUF + b
                pltpu.make_async_copy(
                    table_hbm.at[idx_v.at[g]], bufs[b], sems[b]
                ).wait()
                pltpu.sync_copy(bufs[b], out_hbm.at[pl.ds(base + g * CHUNK, CHUNK)])

                @pl.when(g + NBUF < nchunk)
                def _():
                    pltpu.async_copy(
                        table_hbm.at[idx_v.at[g + NBUF]], bufs[b], sems[b]
                    )

            return carry

        lax.fori_loop(0, nchunk // NBUF, outer, 0)

    return run(x_flat, table)


def kernel(x, token_table):
    batch, seq = x.shape
    vocab, d = token_table.shape
    n_rows = batch * seq
    x_flat = x.reshape(NW, n_rows // NW // CHUNK, CHUNK).astype(jnp.int32)
    out = _sc_embedding_gather(x_flat, token_table, n_rows, d)
    return out.reshape(batch, seq, d)
